# baseline (speedup 1.0000x reference)
import math

import jax
import jax.numpy as jnp
from jax.experimental import pallas as pl
from jax.experimental.pallas import tpu as pltpu


def _wmse_kernel(pred_ref, tgt_ref, w_ref, out_ref, acc_ref):
    i = pl.program_id(1)

    d = pred_ref[...].astype(jnp.float32) - tgt_ref[...].astype(jnp.float32)
    d2 = d * d
    B, T, _ = d2.shape
    part = jnp.sum(d2.reshape(B, T // 8, 8, 128), axis=1)

    @pl.when(i == 0)
    def _first():
        acc_ref[...] = part

    @pl.when(i != 0)
    def _accum():
        acc_ref[...] += part

    @pl.when(i == pl.num_programs(1) - 1)
    def _finalize():
        per_lane = jnp.sum(acc_ref[...], axis=1)
        out_ref[0, ...] = jnp.dot(
            w_ref[...], per_lane, preferred_element_type=jnp.float32
        )


def kernel(predictions, targets, weights):
    orig_shape = predictions.shape
    B = int(orig_shape[0])
    F = int(math.prod(orig_shape[1:])) if len(orig_shape) > 1 else 1

    G = (F + 127) // 128

    pred2 = predictions.reshape(B, F)
    tgt2 = targets.reshape(B, F)
    if G * 128 != F:
        pad = ((0, 0), (0, G * 128 - F))
        pred2 = jnp.pad(pred2, pad)
        tgt2 = jnp.pad(tgt2, pad)
    pred3 = pred2.reshape(B, G, 128)
    tgt3 = tgt2.reshape(B, G, 128)

    n_par = 2 if G % 2 == 0 else 1
    half = G // n_par
    T = None
    for t in (64, 32, 16, 8):
        if half % t == 0:
            T = t
            break
    if T is None:
        T = 8
        half_pad = ((half + T - 1) // T) * T
        extra = half_pad * n_par - G
        pred3 = jnp.pad(pred3, ((0, 0), (0, extra), (0, 0)))
        tgt3 = jnp.pad(tgt3, ((0, 0), (0, extra), (0, 0)))
        half = half_pad
    n_inner = half // T

    w2 = weights.reshape(1, B).astype(jnp.float32)

    partials = pl.pallas_call(
        _wmse_kernel,
        out_shape=jax.ShapeDtypeStruct((n_par, 1, 128), jnp.float32),
        grid=(n_par, n_inner),
        in_specs=[
            pl.BlockSpec((B, T, 128), lambda s, i: (0, s * n_inner + i, 0)),
            pl.BlockSpec((B, T, 128), lambda s, i: (0, s * n_inner + i, 0)),
            pl.BlockSpec((1, B), lambda s, i: (0, 0)),
        ],
        out_specs=pl.BlockSpec((1, 1, 128), lambda s, i: (s, 0, 0)),
        scratch_shapes=[pltpu.VMEM((B, 8, 128), jnp.float32)],
        compiler_params=pltpu.CompilerParams(
            dimension_semantics=("parallel", "arbitrary"),
            vmem_limit_bytes=32 * 1024 * 1024,
        ),
    )(pred3, tgt3, w2)

    total = jnp.sum(partials) * (1.0 / (B * F))
    return total.astype(jnp.float32)


# speedup vs baseline: 1.0457x; 1.0457x over previous
import math

import jax
import jax.numpy as jnp
from jax.experimental import pallas as pl
from jax.experimental.pallas import tpu as pltpu


def _wmse_kernel(pred_ref, tgt_ref, w_ref, out_ref, acc_ref):
    i = pl.program_id(1)

    @pl.when(i == 0)
    def _init():
        acc_ref[...] = jnp.zeros_like(acc_ref)

    d = pred_ref[...].astype(jnp.float32) - tgt_ref[...].astype(jnp.float32)
    d2 = d * d
    B, T, _ = d2.shape
    acc_ref[...] += jnp.sum(d2.reshape(B, T // 8, 8, 128), axis=1)

    @pl.when(i == pl.num_programs(1) - 1)
    def _finalize():
        per_lane = jnp.sum(acc_ref[...], axis=1)
        out_ref[0, ...] = jnp.dot(
            w_ref[...], per_lane, preferred_element_type=jnp.float32
        )


def kernel(predictions, targets, weights):
    orig_shape = predictions.shape
    B = int(orig_shape[0])
    F = int(math.prod(orig_shape[1:])) if len(orig_shape) > 1 else 1

    G = (F + 127) // 128

    pred2 = predictions.reshape(B, F)
    tgt2 = targets.reshape(B, F)
    if G * 128 != F:
        pad = ((0, 0), (0, G * 128 - F))
        pred2 = jnp.pad(pred2, pad)
        tgt2 = jnp.pad(tgt2, pad)
    pred3 = pred2.reshape(B, G, 128)
    tgt3 = tgt2.reshape(B, G, 128)

    n_par = 2 if G % 2 == 0 else 1
    half = G // n_par
    T = None
    for t in (64, 32, 16, 8):
        if half % t == 0:
            T = t
            break
    if T is None:
        T = 8
        half_pad = ((half + T - 1) // T) * T
        extra = half_pad * n_par - G
        pred3 = jnp.pad(pred3, ((0, 0), (0, extra), (0, 0)))
        tgt3 = jnp.pad(tgt3, ((0, 0), (0, extra), (0, 0)))
        half = half_pad
    n_inner = half // T

    w2 = weights.reshape(1, B).astype(jnp.float32)

    partials = pl.pallas_call(
        _wmse_kernel,
        out_shape=jax.ShapeDtypeStruct((n_par, 1, 128), jnp.float32),
        grid=(n_par, n_inner),
        in_specs=[
            pl.BlockSpec((B, T, 128), lambda s, i: (0, s * n_inner + i, 0)),
            pl.BlockSpec((B, T, 128), lambda s, i: (0, s * n_inner + i, 0)),
            pl.BlockSpec((1, B), lambda s, i: (0, 0)),
        ],
        out_specs=pl.BlockSpec((1, 1, 128), lambda s, i: (s, 0, 0)),
        scratch_shapes=[pltpu.VMEM((B, 8, 128), jnp.float32)],
        compiler_params=pltpu.CompilerParams(
            dimension_semantics=("parallel", "arbitrary"),
            vmem_limit_bytes=32 * 1024 * 1024,
        ),
    )(pred3, tgt3, w2)

    total = jnp.sum(partials) * (1.0 / (B * F))
    return total.astype(jnp.float32)
